# baseline (speedup 1.0000x reference)
import functools

import jax
import jax.numpy as jnp
from jax import lax
from jax.experimental import pallas as pl
from jax.experimental.pallas import tpu as pltpu
from jax.experimental.pallas import tpu_sc as plsc

N = 10000
E = 320000
CW = 128
NCH = E // CW
NC, NS = 2, 16
NW = NC * NS
CPW = NCH // NW
NEXTRA = NCH - CPW * NW
RPT = N // NS
D1, D2 = 128, 64
DW = 16
NH = N // 2
DROWS = NH + 8

_MESH = dict(core_axis_name="c", subcore_axis_name="s", num_cores=NC,
             num_subcores=NS)


def _worker_id():
    return lax.axis_index("s") * NC + lax.axis_index("c")


def _chunk_base(wid):
    return CPW * wid + jnp.maximum(wid - (NW - NEXTRA), 0)


def _deg_body(edge_hbm, onesa_hbm, onesb_hbm, zer_hbm, out_hbm,
              dst_v, idxa_v, idxb_v, onesa_v, onesb_v, deg_sh):
    cid = lax.axis_index("c")
    sid = lax.axis_index("s")
    wid = _worker_id()
    base = _chunk_base(wid)
    pltpu.sync_copy(edge_hbm.at[1, pl.ds(base * CW, (CPW + 1) * CW)], dst_v)
    pltpu.sync_copy(onesa_hbm, onesa_v)
    pltpu.sync_copy(onesb_hbm, onesb_v)

    drpt = DROWS // NS
    r0 = sid * drpt
    pltpu.sync_copy(zer_hbm.at[pl.ds(r0, drpt)], deg_sh.at[pl.ds(r0, drpt)])
    plsc.subcore_barrier()

    dump = NH + (lax.iota(jnp.int32, 16) & 7)

    def chunk(j):
        for g in range(CW // 16):
            d = dst_v[pl.ds(j * CW + g * 16, 16)]
            lo = d < NH
            idxa_v[pl.ds(g * 16, 16)] = jnp.where(lo, d, dump)
            idxb_v[pl.ds(g * 16, 16)] = jnp.where(lo, dump, d - NH)
        pltpu.sync_copy(onesa_v, deg_sh.at[idxa_v], add=True)
        pltpu.sync_copy(onesb_v, deg_sh.at[idxb_v], add=True)

    def body(j, _):
        chunk(j)
        return 0

    lax.fori_loop(0, CPW, body, 0)

    @pl.when(wid >= NW - NEXTRA)
    def _():
        chunk(CPW)

    plsc.subcore_barrier()
    pltpu.sync_copy(deg_sh.at[pl.ds(r0, drpt)],
                    out_hbm.at[cid, pl.ds(r0, drpt)])


@functools.cache
def _deg_kernel():
    return functools.partial(
        pl.kernel,
        out_type=jax.ShapeDtypeStruct((NC, DROWS, DW), jnp.float32),
        mesh=plsc.VectorSubcoreMesh(**_MESH),
        compiler_params=pltpu.CompilerParams(use_tc_tiling_on_sc=False),
        scratch_types=[
            pltpu.VMEM(((CPW + 1) * CW,), jnp.int32),
            pltpu.VMEM((CW,), jnp.int32),
            pltpu.VMEM((CW,), jnp.int32),
            pltpu.VMEM((CW, DW), jnp.float32),
            pltpu.VMEM((CW, DW), jnp.float32),
            pltpu.VMEM_SHARED((DROWS, DW), jnp.float32),
        ],
    )(_deg_body)


SLOTS = 2
MAIN = (CPW // SLOTS) * SLOTS


def _agg_body(D, h_hbm, edge_hbm, zer_hbm, out_hbm, src_v, dstb,
              b0, b1, agg_sh, g0, g1, s0, s1, d0, d1):
    bufs = (b0, b1)
    gsem = (g0, g1)
    ssem = (s0, s1)
    dsem = (d0, d1)
    cid = lax.axis_index("c")
    sid = lax.axis_index("s")
    wid = _worker_id()
    base = _chunk_base(wid)
    pltpu.sync_copy(edge_hbm.at[0, pl.ds(base * CW, (CPW + 1) * CW)], src_v)
    r0 = sid * RPT

    @pl.when(cid == 0)
    def _():
        pltpu.sync_copy(h_hbm.at[pl.ds(r0, RPT)], agg_sh.at[pl.ds(r0, RPT)])

    @pl.when(cid == 1)
    def _():
        pltpu.sync_copy(zer_hbm.at[pl.ds(r0, RPT)], agg_sh.at[pl.ds(r0, RPT)])

    plsc.subcore_barrier()

    def gather(j, k):
        pltpu.async_copy(h_hbm.at[src_v.at[pl.ds(j * CW, CW)]], bufs[k],
                         gsem[k])
        pltpu.async_copy(edge_hbm.at[1, pl.ds((base + j) * CW, CW)],
                         dstb.at[k], dsem[k])

    def wait_gather(k):
        pltpu.make_async_copy(h_hbm.at[src_v.at[pl.ds(0, CW)]], bufs[k],
                              gsem[k]).wait()
        pltpu.make_async_copy(edge_hbm.at[1, pl.ds(0, CW)], dstb.at[k],
                              dsem[k]).wait()

    def scatter(k):
        pltpu.async_copy(bufs[k], agg_sh.at[dstb.at[k]], ssem[k], add=True)

    def wait_scatter(k):
        pltpu.make_async_copy(bufs[k], agg_sh.at[dstb.at[k]],
                              ssem[k]).wait()

    for k in range(SLOTS):
        gather(k, k)

    def body(i, _):
        j = SLOTS * i
        for k in range(SLOTS):
            wait_gather(k)
            scatter(k)
        for k in range(SLOTS):
            wait_scatter(k)
            gather(jnp.minimum(j + SLOTS + k, CPW - 1), k)
        return 0

    lax.fori_loop(0, MAIN // SLOTS, body, 0)

    for k in range(SLOTS):
        wait_gather(k)
        if k < CPW - MAIN:
            scatter(k)
            wait_scatter(k)

    @pl.when(wid >= NW - NEXTRA)
    def _():
        pltpu.sync_copy(edge_hbm.at[1, pl.ds((base + CPW) * CW, CW)],
                        dstb.at[0])
        pltpu.sync_copy(h_hbm.at[src_v.at[pl.ds(CPW * CW, CW)]], b0)
        pltpu.sync_copy(b0, agg_sh.at[dstb.at[0]], add=True)
    _ = 0

    plsc.subcore_barrier()
    pltpu.sync_copy(agg_sh.at[pl.ds(r0, RPT)], out_hbm.at[cid, pl.ds(r0, RPT)])


@functools.cache
def _agg_kernel(D):
    return functools.partial(
        pl.kernel,
        out_type=jax.ShapeDtypeStruct((NC, N, D), jnp.float32),
        mesh=plsc.VectorSubcoreMesh(**_MESH),
        compiler_params=pltpu.CompilerParams(use_tc_tiling_on_sc=False),
        scratch_types=[
            pltpu.VMEM(((CPW + 1) * CW,), jnp.int32),
            pltpu.VMEM((SLOTS, CW), jnp.int32),
            pltpu.VMEM((CW, D), jnp.float32),
            pltpu.VMEM((CW, D), jnp.float32),
            pltpu.VMEM_SHARED((N, D), jnp.float32),
        ] + [pltpu.SemaphoreType.DMA] * 6,
    )(functools.partial(_agg_body, D))


def _tc0_body(x_ref, w_ref, u_ref):
    u_ref[...] = jnp.dot(x_ref[...], w_ref[...],
                         preferred_element_type=jnp.float32)


def _tc1_body(u_ref, deg_ref, h_ref, dv_ref):
    dd = deg_ref[...]
    lo = jnp.sum(dd[:, :, 0:8], axis=(0, 2))
    hi = jnp.sum(dd[:, :, 8:16], axis=(0, 2))
    blk = pl.program_id(0)
    d = jnp.where(blk < _GRID // 2, lo, hi)[:, None] + 1.0
    dv = lax.rsqrt(d)
    h_ref[...] = u_ref[...] * dv
    dv_ref[...] = jnp.broadcast_to(dv, (TBLK, 8))


def _tc2_body(a_ref, dv_ref, b_ref, w_ref, o_ref):
    dv = dv_ref[:, 0:1]
    h = jnp.maximum((a_ref[0] + a_ref[1]) * dv + b_ref[...], 0.0)
    o_ref[...] = jnp.dot(h, w_ref[...], preferred_element_type=jnp.float32) * dv


def _tc3_body(a_ref, dv_ref, b_ref, o_ref):
    dv = dv_ref[:, 0:1]
    o_ref[...] = (a_ref[0] + a_ref[1]) * dv + b_ref[...]


TBLK = 1000
_GRID = N // TBLK


def _row_spec(d):
    return pl.BlockSpec((TBLK, d), lambda i: (i, 0))


def _rep_spec(r, c):
    return pl.BlockSpec((r, c), lambda i: (0, 0))


def _agg_spec(d):
    return pl.BlockSpec((NC, TBLK, d), lambda i: (0, i, 0))


_tc0 = pl.pallas_call(
    _tc0_body,
    grid=(_GRID,),
    in_specs=[_row_spec(D1), _rep_spec(D1, D1)],
    out_specs=_row_spec(D1),
    out_shape=jax.ShapeDtypeStruct((N, D1), jnp.float32),
)

_tc1 = pl.pallas_call(
    _tc1_body,
    grid=(_GRID,),
    in_specs=[_row_spec(D1),
              pl.BlockSpec((NC, TBLK, DW), lambda i: (0, i % (_GRID // 2), 0))],
    out_specs=[_row_spec(D1), _row_spec(8)],
    out_shape=[jax.ShapeDtypeStruct((N, D1), jnp.float32),
               jax.ShapeDtypeStruct((N, 8), jnp.float32)],
)

_tc2 = pl.pallas_call(
    _tc2_body,
    grid=(_GRID,),
    in_specs=[_agg_spec(D1), _row_spec(8), _rep_spec(1, D1),
              _rep_spec(D1, D2)],
    out_specs=_row_spec(D2),
    out_shape=jax.ShapeDtypeStruct((N, D2), jnp.float32),
)

_tc3 = pl.pallas_call(
    _tc3_body,
    grid=(_GRID,),
    in_specs=[_agg_spec(D2), _row_spec(8), _rep_spec(1, D2)],
    out_specs=_row_spec(D2),
    out_shape=jax.ShapeDtypeStruct((N, D2), jnp.float32),
)


def kernel(x, edge_index, W1, b1, W2, b2):
    er = edge_index.astype(jnp.int32)

    zdeg = jnp.zeros((DROWS, DW), jnp.float32)
    onesa = jnp.zeros((CW, DW), jnp.float32).at[:, 0].set(1.0)
    onesb = jnp.zeros((CW, DW), jnp.float32).at[:, 8].set(1.0)
    zer1 = jnp.zeros((N, D1), jnp.float32)
    zer2 = jnp.zeros((N, D2), jnp.float32)

    u = _tc0(x, W1)
    deg = _deg_kernel()(er, onesa, onesb, zdeg)
    h1p, dinv = _tc1(u, deg)
    agg1 = _agg_kernel(D1)(h1p, er, zer1)
    h2p = _tc2(agg1, dinv, b1.reshape(1, D1), W2)
    agg2 = _agg_kernel(D2)(h2p, er, zer2)
    return _tc3(agg2, dinv, b2.reshape(1, D2))


# speedup vs baseline: 1.1854x; 1.1854x over previous
import functools

import jax
import jax.numpy as jnp
from jax import lax
from jax.experimental import pallas as pl
from jax.experimental.pallas import tpu as pltpu
from jax.experimental.pallas import tpu_sc as plsc

N = 10000
E = 320000
CW = 128
NCH = E // CW
NC, NS = 2, 16
NW = NC * NS
CPW = NCH // NW
NEXTRA = NCH - CPW * NW
RPT = N // NS
D1, D2 = 128, 64
DW = 16
NH = N // 2
DROWS = NH + 8

_MESH = dict(core_axis_name="c", subcore_axis_name="s", num_cores=NC,
             num_subcores=NS)


def _worker_id():
    return lax.axis_index("s") * NC + lax.axis_index("c")


def _chunk_base(wid):
    return CPW * wid + jnp.maximum(wid - (NW - NEXTRA), 0)


def _deg_body(edge_hbm, onesa_hbm, onesb_hbm, zer_hbm, out_hbm,
              dst_v, idxa_v, idxb_v, onesa_v, onesb_v, deg_sh):
    cid = lax.axis_index("c")
    sid = lax.axis_index("s")
    wid = _worker_id()
    base = _chunk_base(wid)
    pltpu.sync_copy(edge_hbm.at[1, pl.ds(base * CW, (CPW + 1) * CW)], dst_v)
    pltpu.sync_copy(onesa_hbm, onesa_v)
    pltpu.sync_copy(onesb_hbm, onesb_v)

    drpt = DROWS // NS
    r0 = sid * drpt
    pltpu.sync_copy(zer_hbm.at[pl.ds(r0, drpt)], deg_sh.at[pl.ds(r0, drpt)])
    plsc.subcore_barrier()

    dump = NH + (lax.iota(jnp.int32, 16) & 7)

    def chunk(j):
        for g in range(CW // 16):
            d = dst_v[pl.ds(j * CW + g * 16, 16)]
            lo = d < NH
            idxa_v[pl.ds(g * 16, 16)] = jnp.where(lo, d, dump)
            idxb_v[pl.ds(g * 16, 16)] = jnp.where(lo, dump, d - NH)
        pltpu.sync_copy(onesa_v, deg_sh.at[idxa_v], add=True)
        pltpu.sync_copy(onesb_v, deg_sh.at[idxb_v], add=True)

    def body(j, _):
        chunk(j)
        return 0

    lax.fori_loop(0, CPW, body, 0)

    @pl.when(wid >= NW - NEXTRA)
    def _():
        chunk(CPW)

    plsc.subcore_barrier()
    pltpu.sync_copy(deg_sh.at[pl.ds(r0, drpt)],
                    out_hbm.at[cid, pl.ds(r0, drpt)])


@functools.cache
def _deg_kernel():
    return functools.partial(
        pl.kernel,
        out_type=jax.ShapeDtypeStruct((NC, DROWS, DW), jnp.float32),
        mesh=plsc.VectorSubcoreMesh(**_MESH),
        compiler_params=pltpu.CompilerParams(use_tc_tiling_on_sc=False),
        scratch_types=[
            pltpu.VMEM(((CPW + 1) * CW,), jnp.int32),
            pltpu.VMEM((CW,), jnp.int32),
            pltpu.VMEM((CW,), jnp.int32),
            pltpu.VMEM((CW, DW), jnp.float32),
            pltpu.VMEM((CW, DW), jnp.float32),
            pltpu.VMEM_SHARED((DROWS, DW), jnp.float32),
        ],
    )(_deg_body)


SLOTS = 2
MAIN = (CPW // SLOTS) * SLOTS


def _agg_body(D, h_hbm, edge_hbm, zer_hbm, out_hbm, src_v, dstb,
              b0, b1, agg_sh, g0, g1, d0, d1):
    bufs = (b0, b1)
    gsem = (g0, g1)
    dsem = (d0, d1)
    cid = lax.axis_index("c")
    sid = lax.axis_index("s")
    wid = _worker_id()
    base = _chunk_base(wid)
    pltpu.sync_copy(edge_hbm.at[0, pl.ds(base * CW, (CPW + 1) * CW)], src_v)
    r0 = sid * RPT

    @pl.when(cid == 0)
    def _():
        pltpu.sync_copy(h_hbm.at[pl.ds(r0, RPT)], agg_sh.at[pl.ds(r0, RPT)])

    @pl.when(cid == 1)
    def _():
        pltpu.sync_copy(zer_hbm.at[pl.ds(r0, RPT)], agg_sh.at[pl.ds(r0, RPT)])

    plsc.subcore_barrier()

    def gather(j, k):
        pltpu.async_copy(h_hbm.at[src_v.at[pl.ds(j * CW, CW)]], bufs[k],
                         gsem[k])
        pltpu.async_copy(edge_hbm.at[1, pl.ds((base + j) * CW, CW)],
                         dstb.at[k], dsem[k])

    def wait_gather(k):
        pltpu.make_async_copy(h_hbm.at[src_v.at[pl.ds(0, CW)]], bufs[k],
                              gsem[k]).wait()
        pltpu.make_async_copy(edge_hbm.at[1, pl.ds(0, CW)], dstb.at[k],
                              dsem[k]).wait()

    def scatter(k):
        pltpu.sync_copy(bufs[k], agg_sh.at[dstb.at[k]], add=True)

    for k in range(SLOTS):
        gather(k, k)

    def body(i, _):
        j = SLOTS * i
        for k in range(SLOTS):
            wait_gather(k)
            scatter(k)
            gather(jnp.minimum(j + SLOTS + k, CPW - 1), k)
        return 0

    lax.fori_loop(0, MAIN // SLOTS, body, 0)

    for k in range(SLOTS):
        wait_gather(k)
        if k < CPW - MAIN:
            scatter(k)

    @pl.when(wid >= NW - NEXTRA)
    def _():
        pltpu.sync_copy(edge_hbm.at[1, pl.ds((base + CPW) * CW, CW)],
                        dstb.at[0])
        pltpu.sync_copy(h_hbm.at[src_v.at[pl.ds(CPW * CW, CW)]], b0)
        pltpu.sync_copy(b0, agg_sh.at[dstb.at[0]], add=True)
    _ = 0

    plsc.subcore_barrier()
    pltpu.sync_copy(agg_sh.at[pl.ds(r0, RPT)], out_hbm.at[cid, pl.ds(r0, RPT)])


@functools.cache
def _agg_kernel(D):
    return functools.partial(
        pl.kernel,
        out_type=jax.ShapeDtypeStruct((NC, N, D), jnp.float32),
        mesh=plsc.VectorSubcoreMesh(**_MESH),
        compiler_params=pltpu.CompilerParams(use_tc_tiling_on_sc=False),
        scratch_types=[
            pltpu.VMEM(((CPW + 1) * CW,), jnp.int32),
            pltpu.VMEM((SLOTS, CW), jnp.int32),
            pltpu.VMEM((CW, D), jnp.float32),
            pltpu.VMEM((CW, D), jnp.float32),
            pltpu.VMEM_SHARED((N, D), jnp.float32),
        ] + [pltpu.SemaphoreType.DMA] * 4,
    )(functools.partial(_agg_body, D))


def _tc0_body(x_ref, w_ref, u_ref):
    u_ref[...] = jnp.dot(x_ref[...], w_ref[...],
                         preferred_element_type=jnp.float32)


def _tc1_body(u_ref, deg_ref, h_ref, dv_ref):
    dd = deg_ref[...]
    lo = jnp.sum(dd[:, :, 0:8], axis=(0, 2))
    hi = jnp.sum(dd[:, :, 8:16], axis=(0, 2))
    blk = pl.program_id(0)
    d = jnp.where(blk < _GRID // 2, lo, hi)[:, None] + 1.0
    dv = lax.rsqrt(d)
    h_ref[...] = u_ref[...] * dv
    dv_ref[...] = jnp.broadcast_to(dv, (TBLK, 8))


def _tc2_body(a_ref, dv_ref, b_ref, w_ref, o_ref):
    dv = dv_ref[:, 0:1]
    h = jnp.maximum((a_ref[0] + a_ref[1]) * dv + b_ref[...], 0.0)
    o_ref[...] = jnp.dot(h, w_ref[...], preferred_element_type=jnp.float32) * dv


def _tc3_body(a_ref, dv_ref, b_ref, o_ref):
    dv = dv_ref[:, 0:1]
    o_ref[...] = (a_ref[0] + a_ref[1]) * dv + b_ref[...]


TBLK = 1000
_GRID = N // TBLK


def _row_spec(d):
    return pl.BlockSpec((TBLK, d), lambda i: (i, 0))


def _rep_spec(r, c):
    return pl.BlockSpec((r, c), lambda i: (0, 0))


def _agg_spec(d):
    return pl.BlockSpec((NC, TBLK, d), lambda i: (0, i, 0))


_tc0 = pl.pallas_call(
    _tc0_body,
    grid=(_GRID,),
    in_specs=[_row_spec(D1), _rep_spec(D1, D1)],
    out_specs=_row_spec(D1),
    out_shape=jax.ShapeDtypeStruct((N, D1), jnp.float32),
)

_tc1 = pl.pallas_call(
    _tc1_body,
    grid=(_GRID,),
    in_specs=[_row_spec(D1),
              pl.BlockSpec((NC, TBLK, DW), lambda i: (0, i % (_GRID // 2), 0))],
    out_specs=[_row_spec(D1), _row_spec(8)],
    out_shape=[jax.ShapeDtypeStruct((N, D1), jnp.float32),
               jax.ShapeDtypeStruct((N, 8), jnp.float32)],
)

_tc2 = pl.pallas_call(
    _tc2_body,
    grid=(_GRID,),
    in_specs=[_agg_spec(D1), _row_spec(8), _rep_spec(1, D1),
              _rep_spec(D1, D2)],
    out_specs=_row_spec(D2),
    out_shape=jax.ShapeDtypeStruct((N, D2), jnp.float32),
)

_tc3 = pl.pallas_call(
    _tc3_body,
    grid=(_GRID,),
    in_specs=[_agg_spec(D2), _row_spec(8), _rep_spec(1, D2)],
    out_specs=_row_spec(D2),
    out_shape=jax.ShapeDtypeStruct((N, D2), jnp.float32),
)


def kernel(x, edge_index, W1, b1, W2, b2):
    er = edge_index.astype(jnp.int32)

    zdeg = jnp.zeros((DROWS, DW), jnp.float32)
    onesa = jnp.zeros((CW, DW), jnp.float32).at[:, 0].set(1.0)
    onesb = jnp.zeros((CW, DW), jnp.float32).at[:, 8].set(1.0)
    zer1 = jnp.zeros((N, D1), jnp.float32)
    zer2 = jnp.zeros((N, D2), jnp.float32)

    u = _tc0(x, W1)
    deg = _deg_kernel()(er, onesa, onesb, zdeg)
    h1p, dinv = _tc1(u, deg)
    agg1 = _agg_kernel(D1)(h1p, er, zer1)
    h2p = _tc2(agg1, dinv, b1.reshape(1, D1), W2)
    agg2 = _agg_kernel(D2)(h2p, er, zer2)
    return _tc3(agg2, dinv, b2.reshape(1, D2))


# speedup vs baseline: 1.1887x; 1.0027x over previous
import functools

import jax
import jax.numpy as jnp
from jax import lax
from jax.experimental import pallas as pl
from jax.experimental.pallas import tpu as pltpu
from jax.experimental.pallas import tpu_sc as plsc

N = 10000
E = 320000
CW = 128
NCH = E // CW
NC, NS = 2, 16
NW = NC * NS
CPW = NCH // NW
NEXTRA = NCH - CPW * NW
RPT = N // NS
D1, D2 = 128, 64
DW = 16
NH = N // 2
DROWS = NH + 8

_MESH = dict(core_axis_name="c", subcore_axis_name="s", num_cores=NC,
             num_subcores=NS)


def _worker_id():
    return lax.axis_index("s") * NC + lax.axis_index("c")


def _chunk_base(wid):
    return CPW * wid + jnp.maximum(wid - (NW - NEXTRA), 0)


def _deg_body(edge_hbm, onesa_hbm, onesb_hbm, zer_hbm, out_hbm,
              dst_v, idxa_v, idxb_v, onesa_v, onesb_v, deg_sh, sa0, sa1,
              sb0, sb1):
    asem = (sa0, sa1)
    bsem = (sb0, sb1)
    cid = lax.axis_index("c")
    sid = lax.axis_index("s")
    wid = _worker_id()
    base = _chunk_base(wid)
    pltpu.sync_copy(edge_hbm.at[1, pl.ds(base * CW, (CPW + 1) * CW)], dst_v)
    pltpu.sync_copy(onesa_hbm, onesa_v)
    pltpu.sync_copy(onesb_hbm, onesb_v)

    drpt = DROWS // NS
    r0 = sid * drpt
    pltpu.sync_copy(zer_hbm.at[pl.ds(r0, drpt)], deg_sh.at[pl.ds(r0, drpt)])
    plsc.subcore_barrier()

    dump = NH + (lax.iota(jnp.int32, 16) & 7)

    def compute_idx(j, s):
        for g in range(CW // 16):
            d = dst_v[pl.ds(j * CW + g * 16, 16)]
            lo = d < NH
            idxa_v[s, pl.ds(g * 16, 16)] = jnp.where(lo, d, dump)
            idxb_v[s, pl.ds(g * 16, 16)] = jnp.where(lo, dump, d - NH)

    def fire(s):
        pltpu.async_copy(onesa_v, deg_sh.at[idxa_v.at[s]], asem[s], add=True)
        pltpu.async_copy(onesb_v, deg_sh.at[idxb_v.at[s]], bsem[s], add=True)

    def drain(s):
        pltpu.make_async_copy(onesa_v, deg_sh.at[idxa_v.at[s]],
                              asem[s]).wait()
        pltpu.make_async_copy(onesb_v, deg_sh.at[idxb_v.at[s]],
                              bsem[s]).wait()

    compute_idx(0, 0)
    fire(0)
    compute_idx(1, 1)
    fire(1)

    def body(i, _):
        j = 2 * i
        for s in range(2):
            drain(s)
            compute_idx(jnp.minimum(j + 2 + s, CPW - 1), s)

            @pl.when(j + 2 + s <= CPW - 1)
            def _():
                fire(s)
        return 0

    lax.fori_loop(0, CPW // 2 - 1, body, 0)
    drain(0)
    drain(1)

    @pl.when(wid >= NW - NEXTRA)
    def _():
        compute_idx(CPW, 0)
        pltpu.sync_copy(onesa_v, deg_sh.at[idxa_v.at[0]], add=True)
        pltpu.sync_copy(onesb_v, deg_sh.at[idxb_v.at[0]], add=True)

    plsc.subcore_barrier()
    pltpu.sync_copy(deg_sh.at[pl.ds(r0, drpt)],
                    out_hbm.at[cid, pl.ds(r0, drpt)])


@functools.cache
def _deg_kernel():
    return functools.partial(
        pl.kernel,
        out_type=jax.ShapeDtypeStruct((NC, DROWS, DW), jnp.float32),
        mesh=plsc.VectorSubcoreMesh(**_MESH),
        compiler_params=pltpu.CompilerParams(use_tc_tiling_on_sc=False),
        scratch_types=[
            pltpu.VMEM(((CPW + 1) * CW,), jnp.int32),
            pltpu.VMEM((2, CW), jnp.int32),
            pltpu.VMEM((2, CW), jnp.int32),
            pltpu.VMEM((CW, DW), jnp.float32),
            pltpu.VMEM((CW, DW), jnp.float32),
            pltpu.VMEM_SHARED((DROWS, DW), jnp.float32),
        ] + [pltpu.SemaphoreType.DMA] * 4,
    )(_deg_body)


SLOTS = 2
MAIN = (CPW // SLOTS) * SLOTS


def _agg_body(D, h_hbm, edge_hbm, zer_hbm, out_hbm, src_v, dstb,
              b0, b1, agg_sh, g0, g1, d0, d1):
    bufs = (b0, b1)
    gsem = (g0, g1)
    dsem = (d0, d1)
    cid = lax.axis_index("c")
    sid = lax.axis_index("s")
    wid = _worker_id()
    base = _chunk_base(wid)
    pltpu.sync_copy(edge_hbm.at[0, pl.ds(base * CW, (CPW + 1) * CW)], src_v)
    r0 = sid * RPT

    @pl.when(cid == 0)
    def _():
        pltpu.sync_copy(h_hbm.at[pl.ds(r0, RPT)], agg_sh.at[pl.ds(r0, RPT)])

    @pl.when(cid == 1)
    def _():
        pltpu.sync_copy(zer_hbm.at[pl.ds(r0, RPT)], agg_sh.at[pl.ds(r0, RPT)])

    plsc.subcore_barrier()

    def gather(j, k):
        pltpu.async_copy(h_hbm.at[src_v.at[pl.ds(j * CW, CW)]], bufs[k],
                         gsem[k])
        pltpu.async_copy(edge_hbm.at[1, pl.ds((base + j) * CW, CW)],
                         dstb.at[k], dsem[k])

    def wait_gather(k):
        pltpu.make_async_copy(h_hbm.at[src_v.at[pl.ds(0, CW)]], bufs[k],
                              gsem[k]).wait()
        pltpu.make_async_copy(edge_hbm.at[1, pl.ds(0, CW)], dstb.at[k],
                              dsem[k]).wait()

    def scatter(k):
        pltpu.sync_copy(bufs[k], agg_sh.at[dstb.at[k]], add=True)

    for k in range(SLOTS):
        gather(k, k)

    def body(i, _):
        j = SLOTS * i
        for k in range(SLOTS):
            wait_gather(k)
            scatter(k)
            gather(jnp.minimum(j + SLOTS + k, CPW - 1), k)
        return 0

    lax.fori_loop(0, MAIN // SLOTS, body, 0)

    for k in range(SLOTS):
        wait_gather(k)
        if k < CPW - MAIN:
            scatter(k)

    @pl.when(wid >= NW - NEXTRA)
    def _():
        pltpu.sync_copy(edge_hbm.at[1, pl.ds((base + CPW) * CW, CW)],
                        dstb.at[0])
        pltpu.sync_copy(h_hbm.at[src_v.at[pl.ds(CPW * CW, CW)]], b0)
        pltpu.sync_copy(b0, agg_sh.at[dstb.at[0]], add=True)
    _ = 0

    plsc.subcore_barrier()
    pltpu.sync_copy(agg_sh.at[pl.ds(r0, RPT)], out_hbm.at[cid, pl.ds(r0, RPT)])


@functools.cache
def _agg_kernel(D):
    return functools.partial(
        pl.kernel,
        out_type=jax.ShapeDtypeStruct((NC, N, D), jnp.float32),
        mesh=plsc.VectorSubcoreMesh(**_MESH),
        compiler_params=pltpu.CompilerParams(use_tc_tiling_on_sc=False),
        scratch_types=[
            pltpu.VMEM(((CPW + 1) * CW,), jnp.int32),
            pltpu.VMEM((SLOTS, CW), jnp.int32),
            pltpu.VMEM((CW, D), jnp.float32),
            pltpu.VMEM((CW, D), jnp.float32),
            pltpu.VMEM_SHARED((N, D), jnp.float32),
        ] + [pltpu.SemaphoreType.DMA] * 4,
    )(functools.partial(_agg_body, D))


def _tc0_body(x_ref, w_ref, u_ref):
    u_ref[...] = jnp.dot(x_ref[...], w_ref[...],
                         preferred_element_type=jnp.float32)


def _tc1_body(u_ref, deg_ref, h_ref, dv_ref):
    dd = deg_ref[...]
    lo = jnp.sum(dd[:, :, 0:8], axis=(0, 2))
    hi = jnp.sum(dd[:, :, 8:16], axis=(0, 2))
    blk = pl.program_id(0)
    d = jnp.where(blk < _GRID // 2, lo, hi)[:, None] + 1.0
    dv = lax.rsqrt(d)
    h_ref[...] = u_ref[...] * dv
    dv_ref[...] = jnp.broadcast_to(dv, (TBLK, 8))


def _tc2_body(a_ref, dv_ref, b_ref, w_ref, o_ref):
    dv = dv_ref[:, 0:1]
    h = jnp.maximum((a_ref[0] + a_ref[1]) * dv + b_ref[...], 0.0)
    o_ref[...] = jnp.dot(h, w_ref[...], preferred_element_type=jnp.float32) * dv


def _tc3_body(a_ref, dv_ref, b_ref, o_ref):
    dv = dv_ref[:, 0:1]
    o_ref[...] = (a_ref[0] + a_ref[1]) * dv + b_ref[...]


TBLK = 1000
_GRID = N // TBLK


def _row_spec(d):
    return pl.BlockSpec((TBLK, d), lambda i: (i, 0))


def _rep_spec(r, c):
    return pl.BlockSpec((r, c), lambda i: (0, 0))


def _agg_spec(d):
    return pl.BlockSpec((NC, TBLK, d), lambda i: (0, i, 0))


_tc0 = pl.pallas_call(
    _tc0_body,
    grid=(_GRID,),
    in_specs=[_row_spec(D1), _rep_spec(D1, D1)],
    out_specs=_row_spec(D1),
    out_shape=jax.ShapeDtypeStruct((N, D1), jnp.float32),
)

_tc1 = pl.pallas_call(
    _tc1_body,
    grid=(_GRID,),
    in_specs=[_row_spec(D1),
              pl.BlockSpec((NC, TBLK, DW), lambda i: (0, i % (_GRID // 2), 0))],
    out_specs=[_row_spec(D1), _row_spec(8)],
    out_shape=[jax.ShapeDtypeStruct((N, D1), jnp.float32),
               jax.ShapeDtypeStruct((N, 8), jnp.float32)],
)

_tc2 = pl.pallas_call(
    _tc2_body,
    grid=(_GRID,),
    in_specs=[_agg_spec(D1), _row_spec(8), _rep_spec(1, D1),
              _rep_spec(D1, D2)],
    out_specs=_row_spec(D2),
    out_shape=jax.ShapeDtypeStruct((N, D2), jnp.float32),
)

_tc3 = pl.pallas_call(
    _tc3_body,
    grid=(_GRID,),
    in_specs=[_agg_spec(D2), _row_spec(8), _rep_spec(1, D2)],
    out_specs=_row_spec(D2),
    out_shape=jax.ShapeDtypeStruct((N, D2), jnp.float32),
)


def kernel(x, edge_index, W1, b1, W2, b2):
    er = edge_index.astype(jnp.int32)

    zdeg = jnp.zeros((DROWS, DW), jnp.float32)
    onesa = jnp.zeros((CW, DW), jnp.float32).at[:, 0].set(1.0)
    onesb = jnp.zeros((CW, DW), jnp.float32).at[:, 8].set(1.0)
    zer1 = jnp.zeros((N, D1), jnp.float32)
    zer2 = jnp.zeros((N, D2), jnp.float32)

    u = _tc0(x, W1)
    deg = _deg_kernel()(er, onesa, onesb, zdeg)
    h1p, dinv = _tc1(u, deg)
    agg1 = _agg_kernel(D1)(h1p, er, zer1)
    h2p = _tc2(agg1, dinv, b1.reshape(1, D1), W2)
    agg2 = _agg_kernel(D2)(h2p, er, zer2)
    return _tc3(agg2, dinv, b2.reshape(1, D2))
